# baseline (speedup 1.0000x reference)
import jax
import jax.numpy as jnp
from jax.experimental import pallas as pl
from jax.experimental.pallas import tpu as pltpu

_ROWS = 96
_COLS = 16384
_N_CHUNKS = 6
_CHUNK_ROWS = _ROWS // _N_CHUNKS


def _add1_pipeline(x_hbm, o_hbm, buf, in_sems, out_sems):
    for i in range(_N_CHUNKS):
        sl = pl.ds(i * _CHUNK_ROWS, _CHUNK_ROWS)
        pltpu.make_async_copy(x_hbm.at[sl], buf.at[sl], in_sems.at[i]).start()
    for i in range(_N_CHUNKS):
        sl = pl.ds(i * _CHUNK_ROWS, _CHUNK_ROWS)
        pltpu.make_async_copy(x_hbm.at[sl], buf.at[sl], in_sems.at[i]).wait()
        buf[sl] = buf[sl] + 1.0
        pltpu.make_async_copy(buf.at[sl], o_hbm.at[sl], out_sems.at[i]).start()
    for i in range(_N_CHUNKS):
        sl = pl.ds(i * _CHUNK_ROWS, _CHUNK_ROWS)
        pltpu.make_async_copy(buf.at[sl], o_hbm.at[sl], out_sems.at[i]).wait()


def kernel(input_xyzs):
    b, n, c = input_xyzs.shape
    x = jnp.transpose(input_xyzs, (2, 0, 1)).reshape(c * b, n)
    out = pl.pallas_call(
        _add1_pipeline,
        out_shape=jax.ShapeDtypeStruct((c * b, n), jnp.float32),
        in_specs=[pl.BlockSpec(memory_space=pl.ANY)],
        out_specs=pl.BlockSpec(memory_space=pl.ANY),
        scratch_shapes=[
            pltpu.VMEM((_ROWS, _COLS), jnp.float32),
            pltpu.SemaphoreType.DMA((_N_CHUNKS,)),
            pltpu.SemaphoreType.DMA((_N_CHUNKS,)),
        ],
    )(x)
    return jnp.transpose(out.reshape(c, b, n), (1, 2, 0))


# speedup vs baseline: 1.0019x; 1.0019x over previous
import jax
import jax.numpy as jnp
from jax.experimental import pallas as pl
from jax.experimental.pallas import tpu as pltpu

_ROWS = 96
_COLS = 16384
_N_CHUNKS = 12
_CHUNK_ROWS = _ROWS // _N_CHUNKS


def _add1_pipeline(x_hbm, o_hbm, buf, in_sems, out_sems):
    for i in range(_N_CHUNKS):
        sl = pl.ds(i * _CHUNK_ROWS, _CHUNK_ROWS)
        pltpu.make_async_copy(x_hbm.at[sl], buf.at[sl], in_sems.at[i]).start()
    for i in range(_N_CHUNKS):
        sl = pl.ds(i * _CHUNK_ROWS, _CHUNK_ROWS)
        pltpu.make_async_copy(x_hbm.at[sl], buf.at[sl], in_sems.at[i]).wait()
        buf[sl] = buf[sl] + 1.0
        pltpu.make_async_copy(buf.at[sl], o_hbm.at[sl], out_sems.at[i]).start()
    for i in range(_N_CHUNKS):
        sl = pl.ds(i * _CHUNK_ROWS, _CHUNK_ROWS)
        pltpu.make_async_copy(buf.at[sl], o_hbm.at[sl], out_sems.at[i]).wait()


def kernel(input_xyzs):
    b, n, c = input_xyzs.shape
    x = jnp.transpose(input_xyzs, (2, 0, 1)).reshape(c * b, n)
    out = pl.pallas_call(
        _add1_pipeline,
        out_shape=jax.ShapeDtypeStruct((c * b, n), jnp.float32),
        in_specs=[pl.BlockSpec(memory_space=pl.ANY)],
        out_specs=pl.BlockSpec(memory_space=pl.ANY),
        scratch_shapes=[
            pltpu.VMEM((_ROWS, _COLS), jnp.float32),
            pltpu.SemaphoreType.DMA((_N_CHUNKS,)),
            pltpu.SemaphoreType.DMA((_N_CHUNKS,)),
        ],
    )(x)
    return jnp.transpose(out.reshape(c, b, n), (1, 2, 0))
